# Initial kernel scaffold; baseline (speedup 1.0000x reference)
import jax
import jax.numpy as jnp
from jax.experimental import pallas as pl


def kernel(hidden_states, Wg, Wu, Wd, Wsg, Wsu, Wsd, gate_w, shared_gate_w):
    raise NotImplementedError("write your pallas kernel here")


# speedup vs baseline: 1.0207x; 1.0207x over previous
import functools

import jax
import jax.numpy as jnp
from jax import lax
from jax.experimental import pallas as pl
from jax.experimental.pallas import tpu as pltpu

E = 8
TOP_K = 2
H = 1024
FF = 512
T_BLK = 128
NEG = -1e30


def _sigmoid(x):
    return 1.0 / (1.0 + jnp.exp(-x))


def _dense_body(x_ref, gwp_ref, wg_ref, wu_ref, wd_ref, o_ref):
    xb = x_ref[...]
    logits = lax.dot_general(xb, gwp_ref[...], (((1,), (1,)), ((), ())),
                             preferred_element_type=jnp.float32)
    lane = lax.broadcasted_iota(jnp.int32, logits.shape, 1)
    lm = jnp.where(lane < E, logits, NEG)
    m0 = jnp.max(lm, axis=1, keepdims=True)
    a0 = jnp.min(jnp.where(lm == m0, lane, 999), axis=1, keepdims=True)
    lm2 = jnp.where(lane == a0, NEG, lm)
    m1 = jnp.max(lm2, axis=1, keepdims=True)
    a1 = jnp.min(jnp.where(lm2 == m1, lane, 999), axis=1, keepdims=True)
    w0 = _sigmoid(m0 - m1)
    w1 = _sigmoid(m1 - m0)
    sg = _sigmoid(logits[:, E:E + 1])

    def mlp(e):
        hgate = lax.dot_general(xb, wg_ref[e], (((1,), (1,)), ((), ())),
                                preferred_element_type=jnp.float32)
        hup = lax.dot_general(xb, wu_ref[e], (((1,), (1,)), ((), ())),
                              preferred_element_type=jnp.float32)
        hact = hgate * _sigmoid(hgate) * hup
        return lax.dot_general(hact, wd_ref[e], (((1,), (1,)), ((), ())),
                               preferred_element_type=jnp.float32)

    acc = mlp(E) * sg
    for e in range(E):
        cw = w0 * (a0 == e).astype(jnp.float32) + w1 * (a1 == e).astype(jnp.float32)
        acc = acc + mlp(e) * cw
    o_ref[...] = acc


def kernel(hidden_states, Wg, Wu, Wd, Wsg, Wsu, Wsd, gate_w, shared_gate_w):
    B, S, _ = hidden_states.shape
    T = B * S
    x = hidden_states.reshape(T, H)
    wg_ext = jnp.concatenate([Wg, Wsg[None]], axis=0)
    wu_ext = jnp.concatenate([Wu, Wsu[None]], axis=0)
    wd_ext = jnp.concatenate([Wd, Wsd[None]], axis=0)
    gwp = jnp.zeros((128, H), jnp.float32).at[:E].set(gate_w).at[E].set(shared_gate_w[0])

    grid = (T // T_BLK,)
    out = pl.pallas_call(
        _dense_body,
        grid=grid,
        in_specs=[
            pl.BlockSpec((T_BLK, H), lambda g: (g, 0)),
            pl.BlockSpec((128, H), lambda g: (0, 0)),
            pl.BlockSpec((E + 1, FF, H), lambda g: (0, 0, 0)),
            pl.BlockSpec((E + 1, FF, H), lambda g: (0, 0, 0)),
            pl.BlockSpec((E + 1, H, FF), lambda g: (0, 0, 0)),
        ],
        out_specs=pl.BlockSpec((T_BLK, H), lambda g: (g, 0)),
        out_shape=jax.ShapeDtypeStruct((T, H), jnp.float32),
    )(x, gwp, wg_ext, wu_ext, wd_ext)
    return out.reshape(B, S, H)
